# baseline (speedup 1.0000x reference)
import jax
import jax.numpy as jnp
from jax.experimental import pallas as pl


def _add_kernel(x_ref, pe_ref, o_ref):
    o_ref[...] = x_ref[...] + pe_ref[...]


def kernel(x, pe):
    B, N, D = x.shape
    BN = 1024
    nb = N // BN
    return pl.pallas_call(
        _add_kernel,
        grid=(nb, B),
        in_specs=[
            pl.BlockSpec((1, BN, D), lambda n, b: (b, n, 0)),
            pl.BlockSpec((BN, D), lambda n, b: (n, 0)),
        ],
        out_specs=pl.BlockSpec((1, BN, D), lambda n, b: (b, n, 0)),
        out_shape=jax.ShapeDtypeStruct((B, N, D), x.dtype),
    )(x, pe[:N])


# speedup vs baseline: 1.0398x; 1.0398x over previous
import jax
import jax.numpy as jnp
from jax.experimental import pallas as pl


def _add_kernel(nb_pe, bn, x_ref, pe_ref, o_ref):
    i = pl.program_id(0)
    base = (i % nb_pe) * bn
    o_ref[...] = x_ref[...] + pe_ref[pl.ds(base, bn), :]


def kernel(x, pe):
    B, N, D = x.shape
    BN = 1024
    nb_pe = N // BN
    x2 = x.reshape(B * N, D)
    pe_n = pe[:N]
    import functools
    out = pl.pallas_call(
        functools.partial(_add_kernel, nb_pe, BN),
        grid=(B * N // BN,),
        in_specs=[
            pl.BlockSpec((BN, D), lambda i: (i, 0)),
            pl.BlockSpec((N, D), lambda i: (0, 0)),
        ],
        out_specs=pl.BlockSpec((BN, D), lambda i: (i, 0)),
        out_shape=jax.ShapeDtypeStruct((B * N, D), x.dtype),
    )(x2, pe_n)
    return out.reshape(B, N, D)
